# Initial kernel scaffold; baseline (speedup 1.0000x reference)
import jax
import jax.numpy as jnp
from jax.experimental import pallas as pl


def kernel(x, synth_emb_weight):
    raise NotImplementedError("write your pallas kernel here")


# speedup vs baseline: 2.5367x; 2.5367x over previous
import functools

import jax
import jax.numpy as jnp
from jax import lax
from jax.experimental import pallas as pl
from jax.experimental.pallas import tpu as pltpu
from jax.experimental.pallas import tpu_sc as plsc

VOCAB = 7
EMB = 64
PAIR_W = 2 * EMB
LANES = 128
K = 4


def _make_gather(n_pairs):
    info = plsc.get_sparse_core_info()
    nc, ns = info.num_cores, info.num_subcores
    nw = nc * ns
    ir_total = n_pairs // LANES
    ir_per_w = ir_total // nw
    n_chunks = ir_per_w // K
    mesh = plsc.VectorSubcoreMesh(core_axis_name="c", subcore_axis_name="s")

    @functools.partial(
        pl.kernel,
        mesh=mesh,
        out_type=jax.ShapeDtypeStruct((n_pairs, PAIR_W), jnp.float32),
        scratch_types=[
            pltpu.VMEM((K, LANES), jnp.int32),
            pltpu.VMEM((K * LANES, PAIR_W), jnp.float32),
            pltpu.SemaphoreType.DMA,
        ],
    )
    def gather_kernel(table_hbm, idx_hbm, out_hbm, idx_v, rows_v, sem):
        wid = lax.axis_index("s") * nc + lax.axis_index("c")
        w_base = wid * ir_per_w

        def chunk_body(ci, carry):
            ir0 = w_base + ci * K
            pltpu.sync_copy(idx_hbm.at[pl.ds(ir0, K)], idx_v)
            copies = [
                pltpu.async_copy(
                    table_hbm.at[idx_v.at[j]],
                    rows_v.at[pl.ds(j * LANES, LANES)],
                    sem,
                )
                for j in range(K)
            ]
            for c in copies:
                c.wait()
            pltpu.sync_copy(rows_v, out_hbm.at[pl.ds(ir0 * LANES, K * LANES)])
            return carry

        lax.fori_loop(0, n_chunks, chunk_body, 0)

    return gather_kernel


def kernel(x, synth_emb_weight):
    rows, cols = x.shape
    n_pairs = rows * cols // 2
    w = synth_emb_weight
    pair_table = jnp.concatenate(
        [jnp.repeat(w, VOCAB, axis=0), jnp.tile(w, (VOCAB, 1))], axis=1
    )
    xi = x.astype(jnp.int32).reshape(n_pairs, 2)
    pidx = (xi[:, 0] * VOCAB + xi[:, 1]).reshape(n_pairs // LANES, LANES)
    out = _make_gather(n_pairs)(pair_table, pidx)
    return out.reshape(rows, cols * EMB)


# speedup vs baseline: 3.2096x; 1.2652x over previous
import functools

import jax
import jax.numpy as jnp
from jax import lax
from jax.experimental import pallas as pl
from jax.experimental.pallas import tpu as pltpu
from jax.experimental.pallas import tpu_sc as plsc

VOCAB = 7
EMB = 64
PAIR_W = 2 * EMB
IDXW = 64
GROUP = 4
GROWS = GROUP * IDXW


def _make_gather(n_pairs):
    info = plsc.get_sparse_core_info()
    nc, ns = info.num_cores, info.num_subcores
    nw = nc * ns
    pw = n_pairs // nw
    n_ir = pw // IDXW
    n_g = pw // GROWS
    mesh = plsc.VectorSubcoreMesh(core_axis_name="c", subcore_axis_name="s")

    @functools.partial(
        pl.kernel,
        mesh=mesh,
        out_type=jax.ShapeDtypeStruct((n_pairs, PAIR_W), jnp.float32),
        scratch_types=[
            pltpu.VMEM((n_ir, IDXW), jnp.int32),
            pltpu.VMEM((GROWS, PAIR_W), jnp.float32),
            pltpu.VMEM((GROWS, PAIR_W), jnp.float32),
            pltpu.SemaphoreType.DMA,
            pltpu.SemaphoreType.DMA,
        ],
    )
    def gather_kernel(table_hbm, idx_hbm, out_hbm, idx_v, bank_a, bank_b,
                      gsem, ssem):
        wid = lax.axis_index("s") * nc + lax.axis_index("c")
        p0 = wid * pw
        pltpu.sync_copy(idx_hbm.at[pl.ds(wid * n_ir, n_ir)], idx_v)

        def do_group(g, bank):
            copies = [
                pltpu.async_copy(
                    table_hbm.at[idx_v.at[g * GROUP + u]],
                    bank.at[pl.ds(u * IDXW, IDXW)],
                    gsem,
                )
                for u in range(GROUP)
            ]
            for c in copies:
                c.wait()
            pltpu.async_copy(bank, out_hbm.at[pl.ds(p0 + g * GROWS, GROWS)],
                             ssem)

        def drain_scatter(bank):
            pltpu.make_async_copy(
                bank, out_hbm.at[pl.ds(p0, GROWS)], ssem).wait()

        do_group(0, bank_a)
        do_group(1, bank_b)

        def loop_body(g2, carry):
            g = 2 * g2
            drain_scatter(bank_a)
            do_group(g, bank_a)
            drain_scatter(bank_b)
            do_group(g + 1, bank_b)
            return carry

        lax.fori_loop(1, n_g // 2, loop_body, 0)
        drain_scatter(bank_a)
        drain_scatter(bank_b)

    return gather_kernel


def kernel(x, synth_emb_weight):
    rows, cols = x.shape
    n_pairs = rows * cols // 2
    w = synth_emb_weight
    pair_table = jnp.concatenate(
        [jnp.repeat(w, VOCAB, axis=0), jnp.tile(w, (VOCAB, 1))], axis=1
    )
    xi = x.astype(jnp.int32).reshape(n_pairs, 2)
    pidx = (xi[:, 0] * VOCAB + xi[:, 1]).reshape(n_pairs // IDXW, IDXW)
    out = _make_gather(n_pairs)(pair_table, pidx)
    return out.reshape(rows, cols * EMB)


# speedup vs baseline: 5.6519x; 1.7610x over previous
import functools

import jax
import jax.numpy as jnp
from jax import lax
from jax.experimental import pallas as pl
from jax.experimental.pallas import tpu as pltpu
from jax.experimental.pallas import tpu_sc as plsc

VOCAB = 7
EMB = 64
SLAB = 4
SLAB_W = SLAB * EMB
IDXW = 64
GROUP = 2
GROWS = GROUP * IDXW


def _make_gather(n_slabs):
    info = plsc.get_sparse_core_info()
    nc, ns = info.num_cores, info.num_subcores
    nw = nc * ns
    pw = n_slabs // nw
    n_ir = pw // IDXW
    n_g = pw // GROWS
    mesh = plsc.VectorSubcoreMesh(core_axis_name="c", subcore_axis_name="s")

    @functools.partial(
        pl.kernel,
        mesh=mesh,
        out_type=jax.ShapeDtypeStruct((n_slabs, SLAB_W), jnp.float32),
        scratch_types=[
            pltpu.VMEM((n_ir, IDXW), jnp.int32),
            pltpu.VMEM((GROWS, SLAB_W), jnp.float32),
            pltpu.VMEM((GROWS, SLAB_W), jnp.float32),
            pltpu.SemaphoreType.DMA,
            pltpu.SemaphoreType.DMA,
        ],
    )
    def gather_kernel(table_hbm, idx_hbm, out_hbm, idx_v, bank_a, bank_b,
                      gsem, ssem):
        wid = lax.axis_index("s") * nc + lax.axis_index("c")
        p0 = wid * pw
        pltpu.sync_copy(idx_hbm.at[wid], idx_v)

        def do_group(g, bank):
            copies = [
                pltpu.async_copy(
                    table_hbm.at[idx_v.at[g * GROUP + u]],
                    bank.at[pl.ds(u * IDXW, IDXW)],
                    gsem,
                )
                for u in range(GROUP)
            ]
            for c in copies:
                c.wait()
            pltpu.async_copy(bank, out_hbm.at[pl.ds(p0 + g * GROWS, GROWS)],
                             ssem)

        def drain_scatter(bank):
            pltpu.make_async_copy(
                bank, out_hbm.at[pl.ds(p0, GROWS)], ssem).wait()

        do_group(0, bank_a)
        do_group(1, bank_b)

        def loop_body(g2, carry):
            g = 2 * g2
            drain_scatter(bank_a)
            do_group(g, bank_a)
            drain_scatter(bank_b)
            do_group(g + 1, bank_b)
            return carry

        lax.fori_loop(1, n_g // 2, loop_body, 0)
        drain_scatter(bank_a)
        drain_scatter(bank_b)

    return gather_kernel


def kernel(x, synth_emb_weight):
    rows, cols = x.shape
    n_slabs = rows * cols // SLAB
    t = synth_emb_weight
    for _ in range(SLAB.bit_length() - 1):
        v = t.shape[0]
        t = jnp.concatenate([jnp.repeat(t, v, axis=0), jnp.tile(t, (v, 1))],
                            axis=1)
    xi = x.astype(jnp.int32).reshape(n_slabs, SLAB)
    sidx = xi[:, 0]
    for k in range(1, SLAB):
        sidx = sidx * VOCAB + xi[:, k]
    info = plsc.get_sparse_core_info()
    nw = info.num_cores * info.num_subcores
    sidx = sidx.reshape(nw, n_slabs // nw // IDXW, IDXW)
    out = _make_gather(n_slabs)(t, sidx)
    return out.reshape(rows, cols * EMB)
